# baseline (speedup 1.0000x reference)
import jax
import jax.numpy as jnp
from jax import lax
from jax.experimental import pallas as pl
from jax.experimental.pallas import tpu as pltpu

N = 1024
D = 128
OUT = 128
BLK = 1024
JC = 8
K = 24


def _tc_kernel(feat_ref, adj_ref, wemb_ref, bemb_ref, wpool_ref, bpool_ref,
               wsage_ref, bsage_ref, out_ref,
               wm_ref, pow2bf_ref, vals_ref, wmres_ref,
               adjt_ref, aggfb_ref, adj_vmem_ref, adj_sem):
    pid = pl.program_id(0)

    adj_cp = pltpu.make_async_copy(adj_ref, adj_vmem_ref, adj_sem)
    adj_cp.start()

    @pl.when(pid == 0)
    def _():
        wm = jnp.maximum(
            jnp.dot(feat_ref[...], wpool_ref[...],
                    preferred_element_type=jnp.float32) + bpool_ref[...],
            0.0)
        wm_ref[...] = wm
        wmres_ref[...] = wm

        def extract_fast(h, _):
            k0 = 4 * h
            cur = wmres_ref[...]
            for t in range(3):
                mval = jnp.max(cur, axis=0, keepdims=True)
                vals_ref[pl.ds(k0 + t, 1), :] = mval
                cur = jnp.where(cur == mval,
                                -(k0 + t + 2).astype(jnp.float32), cur)
            mval = jnp.max(cur, axis=0, keepdims=True)
            vals_ref[pl.ds(k0 + 3, 1), :] = mval
            wmres_ref[...] = jnp.where(
                cur == mval, -(k0 + 5).astype(jnp.float32), cur)
            return 0

        lax.fori_loop(0, K // 4, extract_fast, 0)

        nmark = jnp.sum((wmres_ref[...] < -1.5).astype(jnp.float32),
                        axis=0, keepdims=True)

        @pl.when(jnp.any(nmark != float(K)))
        def _redo():
            wmres_ref[...] = wm_ref[...]

            def extract_exact(k, _):
                cur = wmres_ref[...]
                jidx = lax.broadcasted_iota(jnp.int32, (N, OUT), 0)
                mval = jnp.max(cur, axis=0, keepdims=True)
                jm = jnp.min(jnp.where(cur == mval, jidx, N),
                             axis=0, keepdims=True)
                vals_ref[pl.ds(k, 1), :] = mval
                wmres_ref[...] = jnp.where(
                    jidx == jm, -(k + 2).astype(jnp.float32), cur)
                return 0

            lax.fori_loop(0, K, extract_exact, 0)

        w = wmres_ref[...]
        ext = w < -1.5
        lab = (-w - 2.0).astype(jnp.int32)
        pa = lax.bitcast_convert_type(((127 + 23) - lab) << 23, jnp.float32)
        pow2bf_ref[...] = jnp.where(ext, pa, 0.0).astype(jnp.bfloat16)
        wmres_ref[...] = jnp.maximum(w, 0.0)

    row0 = pid * BLK
    feat_blk = feat_ref[pl.ds(row0, BLK), :]
    nr = jnp.dot(feat_blk, wemb_ref[...],
                 preferred_element_type=jnp.float32) + bemb_ref[...]

    adj_cp.wait()
    adj_bf = adj_vmem_ref[...].astype(jnp.bfloat16)
    s = lax.dot_general(adj_bf, pow2bf_ref[...],
                        (((1,), (0,)), ((), ())),
                        preferred_element_type=jnp.float32)
    ea = lax.shift_right_logical(lax.bitcast_convert_type(s, jnp.int32), 23)
    kbest = (127 + 23) - ea

    agg = jnp.zeros((BLK, OUT), jnp.float32)
    for ki in range(K):
        v = vals_ref[ki, :].reshape(1, OUT)
        agg = jnp.where(kbest == ki, v, agg)

    resid = kbest >= K
    @pl.when(jnp.any(resid))
    def _():
        ii = lax.broadcasted_iota(jnp.int32, (BLK, BLK), 0)
        bb = lax.broadcasted_iota(jnp.int32, (BLK, BLK), 1)
        eye = (ii == bb).astype(jnp.bfloat16)
        adjt_ref[...] = lax.dot_general(
            adj_bf, eye, (((0,), (0,)), ((), ())),
            preferred_element_type=jnp.float32)

        def jc_body(c, acc):
            a = adjt_ref[pl.ds(c * JC, JC), :]
            w = wmres_ref[pl.ds(c * JC, JC), :]
            cand = jnp.max(a[:, :, None] * w[:, None, :], axis=0)
            return jnp.maximum(acc, cand)
        aggfb_ref[...] = lax.fori_loop(0, N // JC, jc_body,
                                       jnp.zeros((BLK, OUT), jnp.float32))

    agg = jnp.where(resid, aggfb_ref[...], agg)

    w1 = wsage_ref[pl.ds(0, OUT), :]
    w2 = wsage_ref[pl.ds(OUT, OUT), :]
    h = jnp.dot(nr, w1, preferred_element_type=jnp.float32)
    h = h + jnp.dot(agg, w2, preferred_element_type=jnp.float32)
    h = jnp.maximum(h + bsage_ref[...], 0.0)
    sq = jnp.sum(h * h, axis=1, keepdims=True)
    out_ref[...] = h * lax.rsqrt(jnp.maximum(sq, 1e-12))


@jax.jit
def kernel(features, adj_matrix, W_emb, b_emb, W_pool, b_pool, W_sage, b_sage):
    grid = (N // BLK,)
    return pl.pallas_call(
        _tc_kernel,
        grid=grid,
        in_specs=[
            pl.BlockSpec((N, D), lambda i: (0, 0)),
            pl.BlockSpec(memory_space=pl.ANY),
            pl.BlockSpec((D, OUT), lambda i: (0, 0)),
            pl.BlockSpec((1, OUT), lambda i: (0, 0)),
            pl.BlockSpec((D, OUT), lambda i: (0, 0)),
            pl.BlockSpec((1, OUT), lambda i: (0, 0)),
            pl.BlockSpec((2 * OUT, OUT), lambda i: (0, 0)),
            pl.BlockSpec((1, OUT), lambda i: (0, 0)),
        ],
        out_specs=pl.BlockSpec((BLK, OUT), lambda i: (i, 0)),
        out_shape=jax.ShapeDtypeStruct((N, OUT), jnp.float32),
        scratch_shapes=[
            pltpu.VMEM((N, OUT), jnp.float32),
            pltpu.VMEM((N, OUT), jnp.bfloat16),
            pltpu.VMEM((K, OUT), jnp.float32),
            pltpu.VMEM((N, OUT), jnp.float32),
            pltpu.VMEM((N, BLK), jnp.float32),
            pltpu.VMEM((BLK, OUT), jnp.float32),
            pltpu.VMEM((N, N), jnp.int32),
            pltpu.SemaphoreType.DMA,
        ],
    )(features, adj_matrix, W_emb, b_emb.reshape(1, OUT),
      W_pool, b_pool.reshape(1, OUT), W_sage, b_sage.reshape(1, OUT))


# speedup vs baseline: 1.0140x; 1.0140x over previous
import jax
import jax.numpy as jnp
from jax import lax
from jax.experimental import pallas as pl
from jax.experimental.pallas import tpu as pltpu

N = 1024
D = 128
OUT = 128
BLK = 1024
JC = 8
K = 24


def _tc_kernel(feat_ref, adj_ref, wemb_ref, bemb_ref, wpool_ref, bpool_ref,
               wsage_ref, bsage_ref, out_ref,
               wm_ref, pow2bf_ref, vals_ref, wmres_ref,
               adjt_ref, aggfb_ref, adj_vmem_ref, adj_sem):
    pid = pl.program_id(0)

    adj_cp = pltpu.make_async_copy(adj_ref, adj_vmem_ref, adj_sem)
    adj_cp.start()

    @pl.when(pid == 0)
    def _():
        wm = jnp.maximum(
            jnp.dot(feat_ref[...], wpool_ref[...],
                    preferred_element_type=jnp.float32) + bpool_ref[...],
            0.0)
        wm_ref[...] = wm
        wmres_ref[...] = wm

        R = 8

        def extract_fast(h, _):
            k0 = R * h
            cur = wmres_ref[...]
            for t in range(R - 1):
                mval = jnp.max(cur, axis=0, keepdims=True)
                vals_ref[pl.ds(k0 + t, 1), :] = mval
                cur = jnp.where(cur == mval,
                                -(k0 + t + 2).astype(jnp.float32), cur)
            mval = jnp.max(cur, axis=0, keepdims=True)
            vals_ref[pl.ds(k0 + R - 1, 1), :] = mval
            wmres_ref[...] = jnp.where(
                cur == mval, -(k0 + R + 1).astype(jnp.float32), cur)
            return 0

        lax.fori_loop(0, K // R, extract_fast, 0)

        nmark = jnp.sum((wmres_ref[...] < -1.5).astype(jnp.float32),
                        axis=0, keepdims=True)

        @pl.when(jnp.any(nmark != float(K)))
        def _redo():
            wmres_ref[...] = wm_ref[...]

            def extract_exact(k, _):
                cur = wmres_ref[...]
                jidx = lax.broadcasted_iota(jnp.int32, (N, OUT), 0)
                mval = jnp.max(cur, axis=0, keepdims=True)
                jm = jnp.min(jnp.where(cur == mval, jidx, N),
                             axis=0, keepdims=True)
                vals_ref[pl.ds(k, 1), :] = mval
                wmres_ref[...] = jnp.where(
                    jidx == jm, -(k + 2).astype(jnp.float32), cur)
                return 0

            lax.fori_loop(0, K, extract_exact, 0)

        w = wmres_ref[...]
        ext = w < -1.5
        lab = (-w - 2.0).astype(jnp.int32)
        pa = lax.bitcast_convert_type(((127 + 23) - lab) << 23, jnp.float32)
        pow2bf_ref[...] = jnp.where(ext, pa, 0.0).astype(jnp.bfloat16)
        wmres_ref[...] = jnp.maximum(w, 0.0)

    row0 = pid * BLK
    feat_blk = feat_ref[pl.ds(row0, BLK), :]
    nr = jnp.dot(feat_blk, wemb_ref[...],
                 preferred_element_type=jnp.float32) + bemb_ref[...]

    adj_cp.wait()
    adj_bf = adj_vmem_ref[...].astype(jnp.bfloat16)
    s = lax.dot_general(adj_bf, pow2bf_ref[...],
                        (((1,), (0,)), ((), ())),
                        preferred_element_type=jnp.float32)
    ea = lax.shift_right_logical(lax.bitcast_convert_type(s, jnp.int32), 23)
    kbest = (127 + 23) - ea

    agg = jnp.zeros((BLK, OUT), jnp.float32)
    for ki in range(K):
        v = vals_ref[ki, :].reshape(1, OUT)
        agg = jnp.where(kbest == ki, v, agg)

    resid = kbest >= K
    @pl.when(jnp.any(resid))
    def _():
        ii = lax.broadcasted_iota(jnp.int32, (BLK, BLK), 0)
        bb = lax.broadcasted_iota(jnp.int32, (BLK, BLK), 1)
        eye = (ii == bb).astype(jnp.bfloat16)
        adjt_ref[...] = lax.dot_general(
            adj_bf, eye, (((0,), (0,)), ((), ())),
            preferred_element_type=jnp.float32)

        def jc_body(c, acc):
            a = adjt_ref[pl.ds(c * JC, JC), :]
            w = wmres_ref[pl.ds(c * JC, JC), :]
            cand = jnp.max(a[:, :, None] * w[:, None, :], axis=0)
            return jnp.maximum(acc, cand)
        aggfb_ref[...] = lax.fori_loop(0, N // JC, jc_body,
                                       jnp.zeros((BLK, OUT), jnp.float32))

    agg = jnp.where(resid, aggfb_ref[...], agg)

    w1 = wsage_ref[pl.ds(0, OUT), :]
    w2 = wsage_ref[pl.ds(OUT, OUT), :]
    h = jnp.dot(nr, w1, preferred_element_type=jnp.float32)
    h = h + jnp.dot(agg, w2, preferred_element_type=jnp.float32)
    h = jnp.maximum(h + bsage_ref[...], 0.0)
    sq = jnp.sum(h * h, axis=1, keepdims=True)
    out_ref[...] = h * lax.rsqrt(jnp.maximum(sq, 1e-12))


@jax.jit
def kernel(features, adj_matrix, W_emb, b_emb, W_pool, b_pool, W_sage, b_sage):
    grid = (N // BLK,)
    return pl.pallas_call(
        _tc_kernel,
        grid=grid,
        in_specs=[
            pl.BlockSpec((N, D), lambda i: (0, 0)),
            pl.BlockSpec(memory_space=pl.ANY),
            pl.BlockSpec((D, OUT), lambda i: (0, 0)),
            pl.BlockSpec((1, OUT), lambda i: (0, 0)),
            pl.BlockSpec((D, OUT), lambda i: (0, 0)),
            pl.BlockSpec((1, OUT), lambda i: (0, 0)),
            pl.BlockSpec((2 * OUT, OUT), lambda i: (0, 0)),
            pl.BlockSpec((1, OUT), lambda i: (0, 0)),
        ],
        out_specs=pl.BlockSpec((BLK, OUT), lambda i: (i, 0)),
        out_shape=jax.ShapeDtypeStruct((N, OUT), jnp.float32),
        scratch_shapes=[
            pltpu.VMEM((N, OUT), jnp.float32),
            pltpu.VMEM((N, OUT), jnp.bfloat16),
            pltpu.VMEM((K, OUT), jnp.float32),
            pltpu.VMEM((N, OUT), jnp.float32),
            pltpu.VMEM((N, BLK), jnp.float32),
            pltpu.VMEM((BLK, OUT), jnp.float32),
            pltpu.VMEM((N, N), jnp.int32),
            pltpu.SemaphoreType.DMA,
        ],
    )(features, adj_matrix, W_emb, b_emb.reshape(1, OUT),
      W_pool, b_pool.reshape(1, OUT), W_sage, b_sage.reshape(1, OUT))
